# baseline (speedup 1.0000x reference)
import functools

import jax
import jax.numpy as jnp
from jax import lax
from jax.experimental import pallas as pl
from jax.experimental.pallas import tpu as pltpu
from jax.experimental.pallas import tpu_sc as plsc

_POOL = 10
_P = _POOL + 1
_N, _C4, _H, _W = 2, 128, 80, 100
_WP = 128
_K = _H * _W
_C = _C4 // 4
_CPT = 8
_PPT = _CPT // 2
_NBLK = _C4 // _CPT
_NW = 32
_CHUNK = 2000
_NCHUNK = _K // _CHUNK
_G = 16
_NGRP = _CHUNK // _G

_mesh = plsc.VectorSubcoreMesh(core_axis_name="c", subcore_axis_name="s")
_ILV = plsc.PackFormat.INTERLEAVED


@functools.partial(
    pl.kernel,
    out_type=jax.ShapeDtypeStruct((_N * 4 * _C, _K), jnp.float32),
    mesh=_mesh,
    compiler_params=pltpu.CompilerParams(
        needs_layout_passes=False, use_tc_tiling_on_sc=False),
    scratch_types=[
        [pltpu.VMEM((_H * _WP,), jnp.float32)] * _PPT,
        pltpu.VMEM((2, _H, _WP), jnp.float32),
        pltpu.VMEM((_CHUNK, 4), jnp.float32),
        pltpu.VMEM((_CPT, _CHUNK), jnp.float32),
    ],
)
def _border_align_sc(inp_hbm, boxes_hbm, out_hbm, slabs_v, planes_v, box_v,
                     outc_v):
    wid = lax.axis_index("s") * 2 + lax.axis_index("c")
    n = wid // _NBLK
    blk = wid % _NBLK
    border = blk // 4
    c0 = blk * _CPT

    for pr in range(_PPT):
        pltpu.sync_copy(inp_hbm.at[n, pl.ds(c0 + 2 * pr, 2)], planes_v)
        slab = slabs_v[pr]

        def pack_row(r, _, slab=slab):
            base = r * _WP
            for col in range(0, _WP, _G):
                a = planes_v[0, r, pl.ds(col, _G)]
                b = planes_v[1, r, pl.ds(col, _G)]
                slab[pl.ds(base + col, _G)] = plsc.bitcast(
                    plsc.pack(a, b, format=_ILV), jnp.float32)
            return 0

        lax.fori_loop(0, _H, pack_row, 0)

    bsel = jnp.where(border >= 2, jnp.float32(1.0), jnp.float32(0.0))
    ax = (jnp.where(border == 0, jnp.float32(1.0), jnp.float32(0.0))
          - jnp.where(border == 2, jnp.float32(1.0), jnp.float32(0.0)))
    ay = (jnp.where(border == 1, jnp.float32(1.0), jnp.float32(0.0))
          - jnp.where(border == 3, jnp.float32(1.0), jnp.float32(0.0)))
    lanes = lax.iota(jnp.int32, _G)

    def do_group(g, chunk):
        kvec = g * _G + lanes
        zeros = jnp.zeros((_G,), jnp.int32)
        x1 = plsc.load_gather(box_v, [kvec, zeros])
        y1 = plsc.load_gather(box_v, [kvec, zeros + 1])
        x2 = plsc.load_gather(box_v, [kvec, zeros + 2])
        y2 = plsc.load_gather(box_v, [kvec, zeros + 3])
        wx = x2 - x1
        wy = y2 - y1
        dx = wx * (ax * (1.0 / _POOL))
        dy = wy * (ay * (1.0 / _POOL))
        x0 = x1 + wx * bsel
        y0 = y1 + wy * bsel
        m = [None] * _PPT
        for p in range(_P):
            x = jnp.maximum(x0 + jnp.float32(p) * dx, 0.0)
            y = jnp.maximum(y0 + jnp.float32(p) * dy, 0.0)
            xl = x.astype(jnp.int32)
            yl = y.astype(jnp.int32)
            xh = jnp.minimum(xl + 1, _W - 1)
            yh = jnp.minimum(yl + 1, _H - 1)
            lx = jnp.where(xl >= _W - 1, jnp.float32(_W - 1), x) - xl.astype(jnp.float32)
            ly = jnp.where(yl >= _H - 1, jnp.float32(_H - 1), y) - yl.astype(jnp.float32)
            hx = 1.0 - lx
            hy = 1.0 - ly
            w11 = plsc.pack(hy * hx, hy * hx, format=_ILV)
            w12 = plsc.pack(hy * lx, hy * lx, format=_ILV)
            w21 = plsc.pack(ly * hx, ly * hx, format=_ILV)
            w22 = plsc.pack(ly * lx, ly * lx, format=_ILV)
            rl = yl * _WP
            rh = yh * _WP
            i11 = rl + xl
            i12 = rl + xh
            i21 = rh + xl
            i22 = rh + xh
            for pr in range(_PPT):
                sl = slabs_v[pr]
                g11 = plsc.bitcast(plsc.load_gather(sl, [i11]), jnp.bfloat16)
                g12 = plsc.bitcast(plsc.load_gather(sl, [i12]), jnp.bfloat16)
                g21 = plsc.bitcast(plsc.load_gather(sl, [i21]), jnp.bfloat16)
                g22 = plsc.bitcast(plsc.load_gather(sl, [i22]), jnp.bfloat16)
                v = w11 * g11 + w12 * g12 + w21 * g21 + w22 * g22
                m[pr] = v if m[pr] is None else jnp.maximum(m[pr], v)
        for pr in range(_PPT):
            a, b = plsc.unpack(m[pr], format=_ILV)
            outc_v[2 * pr, pl.ds(g * _G, _G)] = a.astype(jnp.float32)
            outc_v[2 * pr + 1, pl.ds(g * _G, _G)] = b.astype(jnp.float32)
        return chunk

    def do_chunk(chunk, carry):
        pltpu.sync_copy(boxes_hbm.at[n, pl.ds(chunk * _CHUNK, _CHUNK)], box_v)
        lax.fori_loop(0, _NGRP, do_group, chunk)
        row0 = n * (4 * _C) + blk * _CPT
        pltpu.sync_copy(outc_v,
                        out_hbm.at[pl.ds(row0, _CPT),
                                   pl.ds(chunk * _CHUNK, _CHUNK)])
        return carry

    lax.fori_loop(0, _NCHUNK, do_chunk, 0)


def kernel(input, boxes):
    inp_p = jnp.pad(input, ((0, 0), (0, 0), (0, 0), (0, _WP - _W)))
    o = _border_align_sc(inp_p, boxes)
    return o.reshape(_N, 4, _C, _K).transpose(0, 2, 3, 1)


# speedup vs baseline: 1.0582x; 1.0582x over previous
import functools

import jax
import jax.numpy as jnp
from jax import lax
from jax.experimental import pallas as pl
from jax.experimental.pallas import tpu as pltpu
from jax.experimental.pallas import tpu_sc as plsc

_POOL = 10
_P = _POOL + 1
_N, _C4, _H, _W = 2, 128, 80, 100
_K = _H * _W
_C = _C4 // 4
_CPT = 8
_PPT = _CPT // 2
_NBLK = _C4 // _CPT
_NW = 32
_CHUNK = 2000
_NCHUNK = _K // _CHUNK
_G = 16
_NGRP = _CHUNK // _G

_mesh = plsc.VectorSubcoreMesh(core_axis_name="c", subcore_axis_name="s")
_ILV = plsc.PackFormat.INTERLEAVED


@functools.partial(
    pl.kernel,
    out_type=jax.ShapeDtypeStruct((_N * 4 * _C // 2, _K), jnp.float32),
    mesh=_mesh,
    compiler_params=pltpu.CompilerParams(
        needs_layout_passes=False, use_tc_tiling_on_sc=False),
    scratch_types=[
        [pltpu.VMEM((_K,), jnp.float32)] * _PPT,
        pltpu.VMEM((2, _H, _W), jnp.float32),
        pltpu.VMEM((4, _K), jnp.float32),
        pltpu.VMEM((_PPT, _CHUNK), jnp.float32),
    ],
)
def _border_align_sc(inp_hbm, boxes_hbm, out_hbm, slabs_v, planes_v, box_v,
                     outc_v):
    wid = lax.axis_index("s") * 2 + lax.axis_index("c")
    n = wid // _NBLK
    blk = wid % _NBLK
    border = blk // 4
    c0 = blk * _CPT

    _COLS = tuple(range(0, _W - _G + 1, _G)) + (_W - _G,)
    for pr in range(_PPT):
        pltpu.sync_copy(inp_hbm.at[n, pl.ds(c0 + 2 * pr, 2)], planes_v)
        slab = slabs_v[pr]

        def pack_row(r, _, slab=slab):
            base = r * _W
            for col in _COLS:
                a = planes_v[0, r, pl.ds(col, _G)]
                b = planes_v[1, r, pl.ds(col, _G)]
                slab[pl.ds(base + col, _G)] = plsc.bitcast(
                    plsc.pack(a, b, format=_ILV), jnp.float32)
            return 0

        lax.fori_loop(0, _H, pack_row, 0)
    pltpu.sync_copy(boxes_hbm.at[n], box_v)

    bsel = jnp.where(border >= 2, jnp.float32(1.0), jnp.float32(0.0))
    ax = (jnp.where(border == 0, jnp.float32(1.0), jnp.float32(0.0))
          - jnp.where(border == 2, jnp.float32(1.0), jnp.float32(0.0)))
    ay = (jnp.where(border == 1, jnp.float32(1.0), jnp.float32(0.0))
          - jnp.where(border == 3, jnp.float32(1.0), jnp.float32(0.0)))

    def do_group(g, chunk):
        kb = chunk * _CHUNK + g * _G
        x1 = box_v[0, pl.ds(kb, _G)]
        y1 = box_v[1, pl.ds(kb, _G)]
        x2 = box_v[2, pl.ds(kb, _G)]
        y2 = box_v[3, pl.ds(kb, _G)]
        wx = x2 - x1
        wy = y2 - y1
        dx = wx * (ax * (1.0 / _POOL))
        dy = wy * (ay * (1.0 / _POOL))
        x0 = x1 + wx * bsel
        y0 = y1 + wy * bsel
        m = [None] * _PPT
        for p in range(_P):
            x = jnp.maximum(x0 + jnp.float32(p) * dx, 0.0)
            y = jnp.maximum(y0 + jnp.float32(p) * dy, 0.0)
            xl = x.astype(jnp.int32)
            yl = y.astype(jnp.int32)
            xh = jnp.minimum(xl + 1, _W - 1)
            yh = jnp.minimum(yl + 1, _H - 1)
            lx = jnp.where(xl >= _W - 1, jnp.float32(_W - 1), x) - xl.astype(jnp.float32)
            ly = jnp.where(yl >= _H - 1, jnp.float32(_H - 1), y) - yl.astype(jnp.float32)
            hx = 1.0 - lx
            hy = 1.0 - ly
            w11 = plsc.pack(hy * hx, hy * hx, format=_ILV)
            w12 = plsc.pack(hy * lx, hy * lx, format=_ILV)
            w21 = plsc.pack(ly * hx, ly * hx, format=_ILV)
            w22 = plsc.pack(ly * lx, ly * lx, format=_ILV)
            rl = yl * _W
            rh = yh * _W
            i11 = rl + xl
            i12 = rl + xh
            i21 = rh + xl
            i22 = rh + xh
            for pr in range(_PPT):
                sl = slabs_v[pr]
                g11 = plsc.bitcast(plsc.load_gather(sl, [i11]), jnp.bfloat16)
                g12 = plsc.bitcast(plsc.load_gather(sl, [i12]), jnp.bfloat16)
                g21 = plsc.bitcast(plsc.load_gather(sl, [i21]), jnp.bfloat16)
                g22 = plsc.bitcast(plsc.load_gather(sl, [i22]), jnp.bfloat16)
                v = w11 * g11 + w12 * g12 + w21 * g21 + w22 * g22
                m[pr] = v if m[pr] is None else jnp.maximum(m[pr], v)
        for pr in range(_PPT):
            outc_v[pr, pl.ds(g * _G, _G)] = plsc.bitcast(m[pr], jnp.float32)
        return chunk

    def do_chunk(chunk, carry):
        lax.fori_loop(0, _NGRP, do_group, chunk)
        row0 = n * (2 * _C) + blk * _PPT
        pltpu.sync_copy(outc_v,
                        out_hbm.at[pl.ds(row0, _PPT),
                                   pl.ds(chunk * _CHUNK, _CHUNK)])
        return carry

    lax.fori_loop(0, _NCHUNK, do_chunk, 0)


def kernel(input, boxes):
    boxes_t = boxes.transpose(0, 2, 1)
    o = _border_align_sc(input, boxes_t)
    ob = lax.bitcast_convert_type(o, jnp.bfloat16)
    ob = ob.reshape(_N, 4, _C // _CPT, _PPT, _K, 2)
    return ob.transpose(0, 2, 3, 5, 4, 1).reshape(
        _N, _C, _K, 4).astype(jnp.float32)


# speedup vs baseline: 1.0870x; 1.0273x over previous
import functools

import jax
import jax.numpy as jnp
from jax import lax
from jax.experimental import pallas as pl
from jax.experimental.pallas import tpu as pltpu
from jax.experimental.pallas import tpu_sc as plsc

_POOL = 10
_P = _POOL + 1
_N, _C4, _H, _W = 2, 128, 80, 100
_K = _H * _W
_C = _C4 // 4
_CPT = 8
_PPT = _CPT // 2
_NBLK = _C4 // _CPT
_NW = 32
_CHUNK = 2000
_NCHUNK = _K // _CHUNK
_G = 16
_NGRP = _CHUNK // _G

_mesh = plsc.VectorSubcoreMesh(core_axis_name="c", subcore_axis_name="s")
_ILV = plsc.PackFormat.INTERLEAVED


@functools.partial(
    pl.kernel,
    out_type=jax.ShapeDtypeStruct((_N * 4 * _C, _K), jnp.float32),
    mesh=_mesh,
    compiler_params=pltpu.CompilerParams(
        needs_layout_passes=False, use_tc_tiling_on_sc=False),
    scratch_types=[
        [pltpu.VMEM((_K,), jnp.float32)] * _PPT,
        [pltpu.VMEM((_H, _W), jnp.float32)] * 2,
        pltpu.VMEM((4, _K), jnp.float32),
        pltpu.VMEM((_CPT, _CHUNK), jnp.float32),
    ],
)
def _border_align_sc(inp_hbm, boxes_hbm, out_hbm, slabs_v, planes_v, box_v,
                     outc_v):
    wid = lax.axis_index("s") * 2 + lax.axis_index("c")
    n = wid // _NBLK
    blk = wid % _NBLK
    border = blk // 4
    c0 = blk * _CPT

    _COLS = tuple(range(0, _W - _G + 1, _G)) + (_W - _G,)
    for pr in range(_PPT):
        pltpu.sync_copy(inp_hbm.at[n, c0 + 2 * pr], planes_v[0])
        pltpu.sync_copy(inp_hbm.at[n, c0 + 2 * pr + 1], planes_v[1])
        slab = slabs_v[pr]

        def pack_row(r, _, slab=slab):
            base = r * _W
            for col in _COLS:
                a = planes_v[0][r, pl.ds(col, _G)]
                b = planes_v[1][r, pl.ds(col, _G)]
                slab[pl.ds(base + col, _G)] = plsc.bitcast(
                    plsc.pack(a, b, format=_ILV), jnp.float32)
            return 0

        lax.fori_loop(0, _H, pack_row, 0)
    pltpu.sync_copy(boxes_hbm.at[n], box_v)

    bsel = jnp.where(border >= 2, jnp.float32(1.0), jnp.float32(0.0))
    ax = (jnp.where(border == 0, jnp.float32(1.0), jnp.float32(0.0))
          - jnp.where(border == 2, jnp.float32(1.0), jnp.float32(0.0)))
    ay = (jnp.where(border == 1, jnp.float32(1.0), jnp.float32(0.0))
          - jnp.where(border == 3, jnp.float32(1.0), jnp.float32(0.0)))

    def do_group(g, chunk):
        kb = chunk * _CHUNK + g * _G
        x1 = box_v[0, pl.ds(kb, _G)]
        y1 = box_v[1, pl.ds(kb, _G)]
        x2 = box_v[2, pl.ds(kb, _G)]
        y2 = box_v[3, pl.ds(kb, _G)]
        wx = x2 - x1
        wy = y2 - y1
        dx = wx * (ax * (1.0 / _POOL))
        dy = wy * (ay * (1.0 / _POOL))
        x0 = x1 + wx * bsel
        y0 = y1 + wy * bsel
        m = [None] * _PPT
        for p in range(_P):
            x = jnp.maximum(x0 + jnp.float32(p) * dx, 0.0)
            y = jnp.maximum(y0 + jnp.float32(p) * dy, 0.0)
            xl = x.astype(jnp.int32)
            yl = y.astype(jnp.int32)
            xh = jnp.minimum(xl + 1, _W - 1)
            yh = jnp.minimum(yl + 1, _H - 1)
            lx = jnp.where(xl >= _W - 1, jnp.float32(_W - 1), x) - xl.astype(jnp.float32)
            ly = jnp.where(yl >= _H - 1, jnp.float32(_H - 1), y) - yl.astype(jnp.float32)
            hx = 1.0 - lx
            hy = 1.0 - ly
            w11 = plsc.pack(hy * hx, hy * hx, format=_ILV)
            w12 = plsc.pack(hy * lx, hy * lx, format=_ILV)
            w21 = plsc.pack(ly * hx, ly * hx, format=_ILV)
            w22 = plsc.pack(ly * lx, ly * lx, format=_ILV)
            rl = yl * _W
            rh = yh * _W
            i11 = rl + xl
            i12 = rl + xh
            i21 = rh + xl
            i22 = rh + xh
            for pr in range(_PPT):
                sl = slabs_v[pr]
                g11 = plsc.bitcast(plsc.load_gather(sl, [i11]), jnp.bfloat16)
                g12 = plsc.bitcast(plsc.load_gather(sl, [i12]), jnp.bfloat16)
                g21 = plsc.bitcast(plsc.load_gather(sl, [i21]), jnp.bfloat16)
                g22 = plsc.bitcast(plsc.load_gather(sl, [i22]), jnp.bfloat16)
                v = w11 * g11 + w12 * g12 + w21 * g21 + w22 * g22
                m[pr] = v if m[pr] is None else jnp.maximum(m[pr], v)
        for pr in range(_PPT):
            a, b = plsc.unpack(m[pr], format=_ILV)
            outc_v[2 * pr, pl.ds(g * _G, _G)] = a.astype(jnp.float32)
            outc_v[2 * pr + 1, pl.ds(g * _G, _G)] = b.astype(jnp.float32)
        return chunk

    def do_chunk(chunk, carry):
        lax.fori_loop(0, _NGRP, do_group, chunk)
        row0 = n * (4 * _C) + blk * _CPT
        pltpu.sync_copy(outc_v,
                        out_hbm.at[pl.ds(row0, _CPT),
                                   pl.ds(chunk * _CHUNK, _CHUNK)])
        return carry

    lax.fori_loop(0, _NCHUNK, do_chunk, 0)


def kernel(input, boxes):
    boxes_t = boxes.transpose(0, 2, 1)
    o = _border_align_sc(input, boxes_t)
    return o.reshape(_N, 4, _C, _K).transpose(0, 2, 3, 1)
